# baseline (speedup 1.0000x reference)
import functools

import jax
import jax.numpy as jnp
import numpy as np
from jax import lax
from jax.experimental import pallas as pl
from jax.experimental.pallas import tpu as pltpu
from jax.experimental.pallas import tpu_sc as plsc

_B, _C, _N, _K, _OUT = 8, 64, 2048, 16, 64
_NBL = 128
_NB = 256
_EPS = 1e-5
_SCALE = float(1 << 21)


def _batcher16():
    n, pairs = 16, []
    p = 1
    while p < n:
        k = p
        while k >= 1:
            for j in range(k % p, n - k, 2 * k):
                for i in range(min(k, n - j - k)):
                    if (i + j) // (2 * p) == (i + j + k) // (2 * p):
                        pairs.append((i + j, i + j + k))
            k //= 2
        p *= 2
    return pairs


_B16 = _batcher16()


def _ce(lst, i, j):
    hi = jnp.maximum(lst[i], lst[j])
    lo = jnp.minimum(lst[i], lst[j])
    lst[i], lst[j] = hi, lo


def _merge_top16(a_lst, b_lst):
    mrg = [jnp.maximum(a_lst[i], b_lst[15 - i]) for i in range(16)]
    for step in (8, 4, 2, 1):
        for i in range(16):
            if (i & step) == 0:
                _ce(mrg, i, i + step)
    return mrg


def _graph_body(xr_ref, xf_ref, a_ref, bm_ref, idx_ref, y1_ref, z_ref):
    xr = xr_ref[0]
    xf = xf_ref[0]
    inner = jax.lax.dot_general(
        xf, xr, (((1,), (1,)), ((), ())), preferred_element_type=jnp.float32)
    xxf = jnp.sum(xf * xf, axis=1, keepdims=True)
    xxr = jnp.sum(xr * xr, axis=1, keepdims=True).T
    d = 2.0 * inner - xxf - xxr
    e = jnp.clip(d, -511.0, 8.0) + 520.0

    q = (e * _SCALE).astype(jnp.int32)
    q3 = q.reshape(256, 8, _NBL)
    slices = [
        jax.lax.bitcast_convert_type(
            (q3[v] & jnp.int32(-256)) | jnp.int32(v), jnp.float32)
        for v in range(256)
    ]

    level = []
    for g in range(16):
        es = [slices[16 * g + t] for t in range(16)]
        for i, j in _B16:
            _ce(es, i, j)
        level.append(es)
    while len(level) > 1:
        level = [_merge_top16(level[i], level[i + 1])
                 for i in range(0, len(level), 2)]
    arrs = level[0]

    sub_iota = jax.lax.broadcasted_iota(jnp.int32, (8, _NBL), 0)
    neg = jnp.float32(-jnp.inf)
    rows = []
    for t in range(_K):
        m8 = arrs[0]
        for i in range(1, 16):
            m8 = jnp.maximum(m8, arrs[i])
        m = jnp.max(m8, axis=0, keepdims=True)
        mb = jnp.broadcast_to(m, (8, _NBL))
        s = jnp.max(jnp.where(m8 == mb, sub_iota, 0), axis=0, keepdims=True)
        for i in range(16):
            arrs[i] = jnp.where(arrs[i] == mb, neg, arrs[i])
        v = jax.lax.bitcast_convert_type(m, jnp.int32) & jnp.int32(255)
        rows.append(((v << 3) | s) + pl.program_id(0) * _N)
    idx_ref[0] = jnp.concatenate(rows, axis=0)

    y1_ref[0] = jax.lax.dot_general(
        xr, a_ref[...], (((1,), (0,)), ((), ())),
        preferred_element_type=jnp.float32)
    z_ref[0] = jax.lax.dot_general(
        xr, bm_ref[...], (((1,), (0,)), ((), ())),
        preferred_element_type=jnp.float32)


def _graph(xs, a_mat, bm_mat):
    nb = xs.shape[0]
    grid = (nb, _N // _NBL)
    return pl.pallas_call(
        _graph_body,
        grid=grid,
        in_specs=[
            pl.BlockSpec((1, _NBL, _C), lambda b, r: (b, r, 0)),
            pl.BlockSpec((1, _N, _C), lambda b, r: (b, 0, 0)),
            pl.BlockSpec((_C, _C), lambda b, r: (0, 0)),
            pl.BlockSpec((_C, _C), lambda b, r: (0, 0)),
        ],
        out_specs=[
            pl.BlockSpec((1, _K, _NBL), lambda b, r: (b, 0, r)),
            pl.BlockSpec((1, _NBL, _C), lambda b, r: (b, r, 0)),
            pl.BlockSpec((1, _NBL, _C), lambda b, r: (b, r, 0)),
        ],
        out_shape=[
            jax.ShapeDtypeStruct((nb, _K, _N), jnp.int32),
            jax.ShapeDtypeStruct((nb, _N, _C), jnp.float32),
            jax.ShapeDtypeStruct((nb, _N, _C), jnp.float32),
        ],
    )(xs, xs, a_mat, bm_mat)


_NW = 32
_ROWS = _B * _K * _N
_RPW = _ROWS // _NW
_CHR = 128
_CH = _RPW // _CHR


def _sc_gather(z_flat, idx2d):
    rows = idx2d.shape[0] * idx2d.shape[1]
    rpw = rows // _NW
    nch = rpw // _CHR
    mesh = plsc.VectorSubcoreMesh(core_axis_name="c", subcore_axis_name="s")

    @functools.partial(
        pl.kernel, mesh=mesh,
        compiler_params=pltpu.CompilerParams(use_tc_tiling_on_sc=False),
        out_type=jax.ShapeDtypeStruct((rows, _C), jnp.float32),
        scratch_types=[
            pltpu.VMEM((nch, _CHR), jnp.int32),
            pltpu.VMEM((_CHR, _C), jnp.float32),
            pltpu.SemaphoreType.DMA,
        ],
    )
    def k(z_hbm, idx_hbm, out_hbm, idx_v, rows_v, sem):
        wid = lax.axis_index("s") * 2 + lax.axis_index("c")
        pltpu.sync_copy(idx_hbm.at[pl.ds(wid * nch, nch)], idx_v)

        def body(j, carry):
            pltpu.async_copy(z_hbm.at[idx_v.at[j]], rows_v, sem).wait()
            pltpu.sync_copy(
                rows_v, out_hbm.at[pl.ds(wid * rpw + j * _CHR, _CHR)])
            return carry

        lax.fori_loop(0, nch, body, 0)

    return k(z_flat, idx2d)


def _stats_body(y1_ref, zg_ref, s1_ref, s2_ref):
    step = pl.program_id(0) * pl.num_programs(1) + pl.program_id(1)

    @pl.when(step == 0)
    def _():
        s1_ref[...] = jnp.zeros_like(s1_ref)
        s2_ref[...] = jnp.zeros_like(s2_ref)

    y1 = y1_ref[0]
    zg = zg_ref[0]
    h = y1[None, :, :] + zg
    s1_ref[...] += jnp.sum(h, axis=(0, 1)).reshape(1, _C)
    s2_ref[...] += jnp.sum(h * h, axis=(0, 1)).reshape(1, _C)


def _stats(y1, zg):
    grid = (y1.shape[0], _N // _NB)
    return pl.pallas_call(
        _stats_body,
        grid=grid,
        in_specs=[
            pl.BlockSpec((1, _NB, _C), lambda b, r: (b, r, 0)),
            pl.BlockSpec((1, _K, _NB, _C), lambda b, r: (b, 0, r, 0)),
        ],
        out_specs=[
            pl.BlockSpec((1, _C), lambda b, r: (0, 0)),
            pl.BlockSpec((1, _C), lambda b, r: (0, 0)),
        ],
        out_shape=[
            jax.ShapeDtypeStruct((1, _C), jnp.float32),
            jax.ShapeDtypeStruct((1, _C), jnp.float32),
        ],
    )(y1, zg)


def _tail_body(y1_ref, zg_ref, sc_ref, sh_ref, w2_ref, o_ref):
    y1 = y1_ref[0]
    zg = zg_ref[0]
    scale = sc_ref[...].reshape(1, 1, _C)
    shift = sh_ref[...].reshape(1, 1, _C)
    h = y1[None, :, :] + zg
    h = jnp.maximum(h * scale + shift, 0.0)
    g = jax.lax.dot_general(
        w2_ref[...], h.reshape(_K * _NB, _C), (((1,), (1,)), ((), ())),
        preferred_element_type=jnp.float32)
    o_ref[0] = jnp.max(g.reshape(_OUT, _K, _NB), axis=1)


def _tail(y1, zg, scale, shift, w2):
    nb = y1.shape[0]
    grid = (nb, _N // _NB)
    return pl.pallas_call(
        _tail_body,
        grid=grid,
        in_specs=[
            pl.BlockSpec((1, _NB, _C), lambda b, r: (b, r, 0)),
            pl.BlockSpec((1, _K, _NB, _C), lambda b, r: (b, 0, r, 0)),
            pl.BlockSpec((1, _C), lambda b, r: (0, 0)),
            pl.BlockSpec((1, _C), lambda b, r: (0, 0)),
            pl.BlockSpec((_OUT, _C), lambda b, r: (0, 0)),
        ],
        out_specs=pl.BlockSpec((1, _OUT, _NB), lambda b, r: (b, 0, r)),
        out_shape=jax.ShapeDtypeStruct((nb, _OUT, _N), jnp.float32),
    )(y1, zg, scale, shift, w2)


def kernel(x, W1, gamma, beta, W2):
    b, c, n = x.shape
    xt = jnp.transpose(x, (0, 2, 1))
    w1a = W1[:, :c]
    w1b = W1[:, c:]

    a2 = (w1a - w1b).T
    b2 = w1b.T

    hb = b // 4
    halves = []
    for xh in (xt[:hb], xt[hb:2 * hb], xt[2 * hb:3 * hb], xt[3 * hb:]):
        idx, y1, z = _graph(xh, a2, b2)
        rows = hb * _K * n
        zg = _sc_gather(z.reshape(hb * n, c),
                        idx.reshape(rows // _CHR, _CHR)).reshape(hb, _K, n, c)
        halves.append((y1, zg))

    parts = [_stats(y1h, zgh) for y1h, zgh in halves]
    s1 = sum(p[0] for p in parts)
    s2 = sum(p[1] for p in parts)
    cnt = float(b * n * _K)
    mean = s1.reshape(-1) / cnt
    var = s2.reshape(-1) / cnt - mean * mean
    rstd = gamma / jnp.sqrt(var + _EPS)
    scale = rstd.reshape(1, -1)
    shift = (beta - mean * rstd).reshape(1, -1)

    outs = [_tail(y1h, zgh, scale, shift, W2) for y1h, zgh in halves]
    return jnp.concatenate(outs, axis=0)


# speedup vs baseline: 1.0403x; 1.0403x over previous
import functools

import jax
import jax.numpy as jnp
import numpy as np
from jax import lax
from jax.experimental import pallas as pl
from jax.experimental.pallas import tpu as pltpu
from jax.experimental.pallas import tpu_sc as plsc

_B, _C, _N, _K, _OUT = 8, 64, 2048, 16, 64
_NBL = 256
_NB = 256
_EPS = 1e-5
_SCALE = float(1 << 21)


def _batcher16():
    n, pairs = 16, []
    p = 1
    while p < n:
        k = p
        while k >= 1:
            for j in range(k % p, n - k, 2 * k):
                for i in range(min(k, n - j - k)):
                    if (i + j) // (2 * p) == (i + j + k) // (2 * p):
                        pairs.append((i + j, i + j + k))
            k //= 2
        p *= 2
    return pairs


_B16 = _batcher16()


def _ce(lst, i, j):
    hi = jnp.maximum(lst[i], lst[j])
    lo = jnp.minimum(lst[i], lst[j])
    lst[i], lst[j] = hi, lo


def _merge_top16(a_lst, b_lst):
    mrg = [jnp.maximum(a_lst[i], b_lst[15 - i]) for i in range(16)]
    for step in (8, 4, 2, 1):
        for i in range(16):
            if (i & step) == 0:
                _ce(mrg, i, i + step)
    return mrg


def _graph_body(xr_ref, xf_ref, a_ref, bm_ref, idx_ref, y1_ref, z_ref):
    xr = xr_ref[0]
    xf = xf_ref[0]
    inner = jax.lax.dot_general(
        xf, xr, (((1,), (1,)), ((), ())), preferred_element_type=jnp.float32)
    xxf = jnp.sum(xf * xf, axis=1, keepdims=True)
    xxr = jnp.sum(xr * xr, axis=1, keepdims=True).T
    d = 2.0 * inner - xxf - xxr
    e = jnp.clip(d, -511.0, 8.0) + 520.0

    q = (e * _SCALE).astype(jnp.int32)
    q3 = q.reshape(256, 8, _NBL)
    slices = [
        jax.lax.bitcast_convert_type(
            (q3[v] & jnp.int32(-256)) | jnp.int32(v), jnp.float32)
        for v in range(256)
    ]

    level = []
    for g in range(16):
        es = [slices[16 * g + t] for t in range(16)]
        for i, j in _B16:
            _ce(es, i, j)
        level.append(es)
    while len(level) > 1:
        level = [_merge_top16(level[i], level[i + 1])
                 for i in range(0, len(level), 2)]
    arrs = level[0]

    sub_iota = jax.lax.broadcasted_iota(jnp.int32, (8, _NBL), 0)
    neg = jnp.float32(-jnp.inf)
    rows = []
    for t in range(_K):
        m8 = arrs[0]
        for i in range(1, 16):
            m8 = jnp.maximum(m8, arrs[i])
        m = jnp.max(m8, axis=0, keepdims=True)
        mb = jnp.broadcast_to(m, (8, _NBL))
        s = jnp.max(jnp.where(m8 == mb, sub_iota, 0), axis=0, keepdims=True)
        for i in range(16):
            arrs[i] = jnp.where(arrs[i] == mb, neg, arrs[i])
        v = jax.lax.bitcast_convert_type(m, jnp.int32) & jnp.int32(255)
        rows.append(((v << 3) | s) + pl.program_id(0) * _N)
    idx_ref[0] = jnp.concatenate(rows, axis=0)

    y1_ref[0] = jax.lax.dot_general(
        xr, a_ref[...], (((1,), (0,)), ((), ())),
        preferred_element_type=jnp.float32)
    z_ref[0] = jax.lax.dot_general(
        xr, bm_ref[...], (((1,), (0,)), ((), ())),
        preferred_element_type=jnp.float32)


def _graph(xs, a_mat, bm_mat):
    nb = xs.shape[0]
    grid = (nb, _N // _NBL)
    return pl.pallas_call(
        _graph_body,
        grid=grid,
        in_specs=[
            pl.BlockSpec((1, _NBL, _C), lambda b, r: (b, r, 0)),
            pl.BlockSpec((1, _N, _C), lambda b, r: (b, 0, 0)),
            pl.BlockSpec((_C, _C), lambda b, r: (0, 0)),
            pl.BlockSpec((_C, _C), lambda b, r: (0, 0)),
        ],
        out_specs=[
            pl.BlockSpec((1, _K, _NBL), lambda b, r: (b, 0, r)),
            pl.BlockSpec((1, _NBL, _C), lambda b, r: (b, r, 0)),
            pl.BlockSpec((1, _NBL, _C), lambda b, r: (b, r, 0)),
        ],
        out_shape=[
            jax.ShapeDtypeStruct((nb, _K, _N), jnp.int32),
            jax.ShapeDtypeStruct((nb, _N, _C), jnp.float32),
            jax.ShapeDtypeStruct((nb, _N, _C), jnp.float32),
        ],
    )(xs, xs, a_mat, bm_mat)


_NW = 32
_ROWS = _B * _K * _N
_RPW = _ROWS // _NW
_CHR = 128
_CH = _RPW // _CHR


def _sc_gather(z_flat, idx2d):
    rows = idx2d.shape[0] * idx2d.shape[1]
    rpw = rows // _NW
    nch = rpw // _CHR
    mesh = plsc.VectorSubcoreMesh(core_axis_name="c", subcore_axis_name="s")

    @functools.partial(
        pl.kernel, mesh=mesh,
        compiler_params=pltpu.CompilerParams(use_tc_tiling_on_sc=False),
        out_type=jax.ShapeDtypeStruct((rows, _C), jnp.float32),
        scratch_types=[
            pltpu.VMEM((nch, _CHR), jnp.int32),
            pltpu.VMEM((_CHR, _C), jnp.float32),
            pltpu.SemaphoreType.DMA,
        ],
    )
    def k(z_hbm, idx_hbm, out_hbm, idx_v, rows_v, sem):
        wid = lax.axis_index("s") * 2 + lax.axis_index("c")
        pltpu.sync_copy(idx_hbm.at[pl.ds(wid * nch, nch)], idx_v)

        def body(j, carry):
            pltpu.async_copy(z_hbm.at[idx_v.at[j]], rows_v, sem).wait()
            pltpu.sync_copy(
                rows_v, out_hbm.at[pl.ds(wid * rpw + j * _CHR, _CHR)])
            return carry

        lax.fori_loop(0, nch, body, 0)

    return k(z_flat, idx2d)


def _stats_body(y1_ref, zg_ref, s1_ref, s2_ref):
    step = pl.program_id(0) * pl.num_programs(1) + pl.program_id(1)

    @pl.when(step == 0)
    def _():
        s1_ref[...] = jnp.zeros_like(s1_ref)
        s2_ref[...] = jnp.zeros_like(s2_ref)

    y1 = y1_ref[0]
    zg = zg_ref[0]
    h = y1[None, :, :] + zg
    s1_ref[...] += jnp.sum(h, axis=(0, 1)).reshape(1, _C)
    s2_ref[...] += jnp.sum(h * h, axis=(0, 1)).reshape(1, _C)


def _stats(y1, zg):
    grid = (y1.shape[0], _N // _NB)
    return pl.pallas_call(
        _stats_body,
        grid=grid,
        in_specs=[
            pl.BlockSpec((1, _NB, _C), lambda b, r: (b, r, 0)),
            pl.BlockSpec((1, _K, _NB, _C), lambda b, r: (b, 0, r, 0)),
        ],
        out_specs=[
            pl.BlockSpec((1, _C), lambda b, r: (0, 0)),
            pl.BlockSpec((1, _C), lambda b, r: (0, 0)),
        ],
        out_shape=[
            jax.ShapeDtypeStruct((1, _C), jnp.float32),
            jax.ShapeDtypeStruct((1, _C), jnp.float32),
        ],
    )(y1, zg)


def _tail_body(y1_ref, zg_ref, sc_ref, sh_ref, w2_ref, o_ref):
    y1 = y1_ref[0]
    zg = zg_ref[0]
    scale = sc_ref[...].reshape(1, 1, _C)
    shift = sh_ref[...].reshape(1, 1, _C)
    h = y1[None, :, :] + zg
    h = jnp.maximum(h * scale + shift, 0.0)
    g = jax.lax.dot_general(
        w2_ref[...], h.reshape(_K * _NB, _C), (((1,), (1,)), ((), ())),
        preferred_element_type=jnp.float32)
    o_ref[0] = jnp.max(g.reshape(_OUT, _K, _NB), axis=1)


def _tail(y1, zg, scale, shift, w2):
    nb = y1.shape[0]
    grid = (nb, _N // _NB)
    return pl.pallas_call(
        _tail_body,
        grid=grid,
        in_specs=[
            pl.BlockSpec((1, _NB, _C), lambda b, r: (b, r, 0)),
            pl.BlockSpec((1, _K, _NB, _C), lambda b, r: (b, 0, r, 0)),
            pl.BlockSpec((1, _C), lambda b, r: (0, 0)),
            pl.BlockSpec((1, _C), lambda b, r: (0, 0)),
            pl.BlockSpec((_OUT, _C), lambda b, r: (0, 0)),
        ],
        out_specs=pl.BlockSpec((1, _OUT, _NB), lambda b, r: (b, 0, r)),
        out_shape=jax.ShapeDtypeStruct((nb, _OUT, _N), jnp.float32),
    )(y1, zg, scale, shift, w2)


def kernel(x, W1, gamma, beta, W2):
    b, c, n = x.shape
    xt = jnp.transpose(x, (0, 2, 1))
    w1a = W1[:, :c]
    w1b = W1[:, c:]

    a2 = (w1a - w1b).T
    b2 = w1b.T

    hb = b // 2
    halves = []
    for xh in (xt[:hb], xt[hb:]):
        idx, y1, z = _graph(xh, a2, b2)
        rows = hb * _K * n
        zg = _sc_gather(z.reshape(hb * n, c),
                        idx.reshape(rows // _CHR, _CHR)).reshape(hb, _K, n, c)
        halves.append((y1, zg))

    parts = [_stats(y1h, zgh) for y1h, zgh in halves]
    s1 = sum(p[0] for p in parts)
    s2 = sum(p[1] for p in parts)
    cnt = float(b * n * _K)
    mean = s1.reshape(-1) / cnt
    var = s2.reshape(-1) / cnt - mean * mean
    rstd = gamma / jnp.sqrt(var + _EPS)
    scale = rstd.reshape(1, -1)
    shift = (beta - mean * rstd).reshape(1, -1)

    outs = [_tail(y1h, zgh, scale, shift, W2) for y1h, zgh in halves]
    return jnp.concatenate(outs, axis=0)
